# baseline (speedup 1.0000x reference)
import functools

import jax
import jax.numpy as jnp
from jax import lax
from jax.experimental import pallas as pl
from jax.experimental.pallas import tpu as pltpu
from jax.experimental.pallas import tpu_sc as plsc

_VOCAB = 100000
_EMBED_DIM = 128
_N_ROWS = 64
_STRIDE = 1562
_LANES = 16
_N_WORKERS = _N_ROWS // _LANES


def _sc_gather(x):
    mesh = plsc.VectorSubcoreMesh(core_axis_name="c", subcore_axis_name="s")

    @functools.partial(
        pl.kernel,
        mesh=mesh,
        out_type=jax.ShapeDtypeStruct((_N_ROWS, _EMBED_DIM), jnp.float32),
        scratch_types=[
            pltpu.VMEM((_LANES,), jnp.int32),
            pltpu.VMEM((_LANES, _EMBED_DIM), jnp.float32),
            pltpu.SemaphoreType.DMA,
        ],
    )
    def k(x_hbm, out_hbm, idx_v, rows_v, sem):
        wid = lax.axis_index("s") * 2 + lax.axis_index("c")

        @pl.when(wid < _N_WORKERS)
        def _():
            lanes = lax.iota(jnp.int32, _LANES)
            idx_v[...] = (wid * _LANES + lanes) * _STRIDE
            pltpu.async_copy(x_hbm.at[idx_v], rows_v, sem).wait()
            pltpu.sync_copy(rows_v, out_hbm.at[pl.ds(wid * _LANES, _LANES)])

    return k(x)


def kernel(x):
    return _sc_gather(x)


# speedup vs baseline: 1.0694x; 1.0694x over previous
import functools

import jax
import jax.numpy as jnp
from jax import lax
from jax.experimental import pallas as pl
from jax.experimental.pallas import tpu as pltpu
from jax.experimental.pallas import tpu_sc as plsc

_VOCAB = 100000
_EMBED_DIM = 128
_N_ROWS = 64
_STRIDE = 1562
_LANES = 16
_N_WORKERS = _N_ROWS // _LANES


def _sc_gather(x):
    mesh = plsc.VectorSubcoreMesh(
        core_axis_name="c", subcore_axis_name="s", num_cores=1
    )

    @functools.partial(
        pl.kernel,
        mesh=mesh,
        out_type=jax.ShapeDtypeStruct((_N_ROWS, _EMBED_DIM), jnp.float32),
        scratch_types=[
            pltpu.VMEM((_LANES,), jnp.int32),
            pltpu.VMEM((_LANES, _EMBED_DIM), jnp.float32),
            pltpu.SemaphoreType.DMA,
        ],
    )
    def k(x_hbm, out_hbm, idx_v, rows_v, sem):
        wid = lax.axis_index("s")

        @pl.when(wid < _N_WORKERS)
        def _():
            lanes = lax.iota(jnp.int32, _LANES)
            idx_v[...] = (wid * _LANES + lanes) * _STRIDE
            pltpu.async_copy(x_hbm.at[idx_v], rows_v, sem).wait()
            pltpu.sync_copy(rows_v, out_hbm.at[pl.ds(wid * _LANES, _LANES)])

    return k(x)


def kernel(x):
    return _sc_gather(x)


# speedup vs baseline: 1.0697x; 1.0002x over previous
import functools

import jax
import jax.numpy as jnp
from jax import lax
from jax.experimental import pallas as pl
from jax.experimental.pallas import tpu as pltpu
from jax.experimental.pallas import tpu_sc as plsc

_VOCAB = 100000
_EMBED_DIM = 128
_N_ROWS = 64
_STRIDE = 1562
_LANES = 16
_N_WORKERS = _N_ROWS // _LANES


def _sc_gather(x):
    mesh = plsc.VectorSubcoreMesh(
        core_axis_name="c", subcore_axis_name="s", num_cores=1
    )

    @functools.partial(
        pl.kernel,
        mesh=mesh,
        out_type=jax.ShapeDtypeStruct((_N_ROWS, _EMBED_DIM), jnp.float32),
        scratch_types=[
            pltpu.VMEM((_LANES, _EMBED_DIM), jnp.float32),
            pltpu.SemaphoreType.DMA,
        ],
    )
    def k(x_hbm, out_hbm, rows_v, sem):
        wid = lax.axis_index("s")

        @pl.when(wid < _N_WORKERS)
        def _():
            lanes = lax.iota(jnp.int32, _LANES)
            idx = (wid * _LANES + lanes) * _STRIDE
            pltpu.async_copy(x_hbm.at[idx], rows_v, sem).wait()
            pltpu.sync_copy(rows_v, out_hbm.at[pl.ds(wid * _LANES, _LANES)])

    return k(x)


def kernel(x):
    return _sc_gather(x)
